# baseline (speedup 1.0000x reference)
import functools

import jax
import jax.numpy as jnp
from jax import lax
from jax.experimental import pallas as pl
from jax.experimental.pallas import tpu as pltpu
from jax.experimental.pallas import tpu_sc as plsc

L = 16
NC = 2
NS = 16
NW = NC * NS
B, C, H, W = 16, 3, 512, 512
PLANE = H * W
W2D = 512
T = 4096
SLOTS = 4
U = 4
THRESH = 0.01

K_SC = 6
M_SC = K_SC * PLANE
CHUNK = M_SC // NW
N_SUB = CHUNK // T
RT = T // W2D

_SCRATCH = (
    [pltpu.VMEM((RT, W2D), jnp.float32) for _ in range(7 * SLOTS)]
    + [pltpu.VMEM((L,), jnp.float32) for _ in range(2)]
    + [pltpu.SemaphoreType.DMA for _ in range(SLOTS)]
)


@functools.partial(
    pl.kernel,
    out_type=jax.ShapeDtypeStruct((2, NW, L), jnp.float32),
    mesh=plsc.VectorSubcoreMesh(core_axis_name="c", subcore_axis_name="s"),
    scratch_types=_SCRATCH,
)
def _sc_partials(in_hbm, tg_hbm, mk_hbm, out_hbm, *s):
    bufs = tuple(s[7 * k:7 * (k + 1)] for k in range(SLOTS))
    sum_b, cnt_b = s[7 * SLOTS], s[7 * SLOTS + 1]
    sems = s[7 * SLOTS + 2:7 * SLOTS + 2 + SLOTS]

    cid = lax.axis_index("c")
    sid = lax.axis_index("s")
    wid = sid * NC + cid
    mb = wid * CHUNK

    def issue(j, slot):
        sem = sems[slot]
        mk_b, i0, i1, i2, t0, t1, t2 = bufs[slot]
        moff = mb + j * T
        bb = moff // PLANE
        q = moff % PLANE
        mrow = pl.multiple_of(moff // W2D, 8)
        descs = [pltpu.async_copy(mk_hbm.at[pl.ds(mrow, RT), :], mk_b, sem)]
        for c, (ib, tb) in enumerate(((i0, t0), (i1, t1), (i2, t2))):
            row = pl.multiple_of(((bb * C + c) * PLANE + q) // W2D, 8)
            descs.append(pltpu.async_copy(in_hbm.at[pl.ds(row, RT), :], ib, sem))
            descs.append(pltpu.async_copy(tg_hbm.at[pl.ds(row, RT), :], tb, sem))
        return descs

    z = jnp.zeros((L,), jnp.float32)
    carry0 = (z,) * 8
    pending = {}
    for j in range(min(SLOTS - 1, N_SUB)):
        pending[j] = issue(j, j % SLOTS)
    for j in range(N_SUB):
        slot = j % SLOTS
        for dsc in pending.pop(j):
            dsc.wait()
        nxt = j + SLOTS - 1
        if nxt < N_SUB:
            pending[nxt] = issue(nxt, nxt % SLOTS)

        def row_loop(r, carry, _bufs=bufs[slot]):
            def step(g, carry2):
                accs = list(carry2)
                mk_v, a0, a1, a2, b0, b1, b2 = _bufs
                for u in range(U):
                    p = u % 2
                    sl = pl.ds((g * U + u) * L, L)
                    m = mk_v[r, sl] > THRESH
                    accs[6 + p] = accs[6 + p] + jnp.where(m, 1.0, 0.0)
                    for c, (av, bv) in enumerate(((a0, b0), (a1, b1), (a2, b2))):
                        d = jnp.abs(av[r, sl] - bv[r, sl])
                        accs[2 * c + p] = accs[2 * c + p] + jnp.where(m, d, 0.0)
                return tuple(accs)

            return lax.fori_loop(0, W2D // (L * U), step, carry)

        carry0 = lax.fori_loop(0, RT, row_loop, carry0)

    sum_b[...] = (carry0[0] + carry0[1]) + (carry0[2] + carry0[3]) \
        + (carry0[4] + carry0[5])
    cnt_b[...] = carry0[6] + carry0[7]
    pltpu.sync_copy(sum_b, out_hbm.at[0, wid])
    pltpu.sync_copy(cnt_b, out_hbm.at[1, wid])


BB = 2


def _tc_body(in_ref, tg_ref, mk_ref, out_ref):
    g = pl.program_id(0)
    psum = jnp.zeros((8, W), jnp.float32)
    pcnt = jnp.zeros((8, W), jnp.float32)
    for k in range(BB):
        mf = jnp.where(mk_ref[k, 0] > THRESH, 1.0, 0.0)
        a, t = in_ref[k], tg_ref[k]
        dsum = (jnp.abs(a[0] - t[0]) + jnp.abs(a[1] - t[1])) \
            + jnp.abs(a[2] - t[2])
        masked = dsum * mf
        psum = psum + jnp.sum(masked.reshape(H // 8, 8, W), axis=0)
        pcnt = pcnt + jnp.sum(mf.reshape(H // 8, 8, W), axis=0)

    @pl.when(g == 0)
    def _init():
        out_ref[0] = psum
        out_ref[1] = pcnt

    @pl.when(g > 0)
    def _accum():
        out_ref[0] += psum
        out_ref[1] += pcnt


def _tc_partials(input, target, mask):
    return pl.pallas_call(
        _tc_body,
        grid=((B - K_SC) // BB,),
        in_specs=[
            pl.BlockSpec((BB, C, H, W), lambda g: (g + K_SC // BB, 0, 0, 0)),
            pl.BlockSpec((BB, C, H, W), lambda g: (g + K_SC // BB, 0, 0, 0)),
            pl.BlockSpec((BB, 1, H, W), lambda g: (g + K_SC // BB, 0, 0, 0)),
        ],
        out_specs=pl.BlockSpec((2, 8, W), lambda g: (0, 0, 0)),
        out_shape=jax.ShapeDtypeStruct((2, 8, W), jnp.float32),
    )(input, target, mask)


def kernel(input, target, mask):
    sc = _sc_partials(
        input.reshape(B * C * H, W),
        target.reshape(B * C * H, W),
        mask.reshape(B * H, W))
    tc = _tc_partials(input, target, mask)
    sel_sum = jnp.sum(sc[0]) + jnp.sum(tc[0])
    count = C * (jnp.sum(sc[1]) + jnp.sum(tc[1]))
    return sel_sum / jnp.maximum(count, 1.0)


# speedup vs baseline: 1.0150x; 1.0150x over previous
import functools

import jax
import jax.numpy as jnp
from jax import lax
from jax.experimental import pallas as pl
from jax.experimental.pallas import tpu as pltpu
from jax.experimental.pallas import tpu_sc as plsc

L = 16
NC = 2
NS = 16
NW = NC * NS
B, C, H, W = 16, 3, 512, 512
PLANE = H * W
W2D = 512
T = 4096
SLOTS = 4
U = 4
THRESH = 0.01

K_SC = 4
M_SC = K_SC * PLANE
CHUNK = M_SC // NW
N_SUB = CHUNK // T
RT = T // W2D

_SCRATCH = (
    [pltpu.VMEM((RT, W2D), jnp.float32) for _ in range(7 * SLOTS)]
    + [pltpu.VMEM((L,), jnp.float32) for _ in range(2)]
    + [pltpu.SemaphoreType.DMA for _ in range(SLOTS)]
)


@functools.partial(
    pl.kernel,
    out_type=jax.ShapeDtypeStruct((2, NW, L), jnp.float32),
    mesh=plsc.VectorSubcoreMesh(core_axis_name="c", subcore_axis_name="s"),
    scratch_types=_SCRATCH,
)
def _sc_partials(in_hbm, tg_hbm, mk_hbm, out_hbm, *s):
    bufs = tuple(s[7 * k:7 * (k + 1)] for k in range(SLOTS))
    sum_b, cnt_b = s[7 * SLOTS], s[7 * SLOTS + 1]
    sems = s[7 * SLOTS + 2:7 * SLOTS + 2 + SLOTS]

    cid = lax.axis_index("c")
    sid = lax.axis_index("s")
    wid = sid * NC + cid
    mb = wid * CHUNK

    def issue(j, slot):
        sem = sems[slot]
        mk_b, i0, i1, i2, t0, t1, t2 = bufs[slot]
        moff = mb + j * T
        bb = moff // PLANE
        q = moff % PLANE
        mrow = pl.multiple_of(moff // W2D, 8)
        descs = [pltpu.async_copy(mk_hbm.at[pl.ds(mrow, RT), :], mk_b, sem)]
        for c, (ib, tb) in enumerate(((i0, t0), (i1, t1), (i2, t2))):
            row = pl.multiple_of(((bb * C + c) * PLANE + q) // W2D, 8)
            descs.append(pltpu.async_copy(in_hbm.at[pl.ds(row, RT), :], ib, sem))
            descs.append(pltpu.async_copy(tg_hbm.at[pl.ds(row, RT), :], tb, sem))
        return descs

    z = jnp.zeros((L,), jnp.float32)
    carry0 = (z,) * 8
    pending = {}
    for j in range(min(SLOTS - 1, N_SUB)):
        pending[j] = issue(j, j % SLOTS)
    for j in range(N_SUB):
        slot = j % SLOTS
        for dsc in pending.pop(j):
            dsc.wait()
        nxt = j + SLOTS - 1
        if nxt < N_SUB:
            pending[nxt] = issue(nxt, nxt % SLOTS)

        def row_loop(r, carry, _bufs=bufs[slot]):
            def step(g, carry2):
                accs = list(carry2)
                mk_v, a0, a1, a2, b0, b1, b2 = _bufs
                for u in range(U):
                    p = u % 2
                    sl = pl.ds((g * U + u) * L, L)
                    m = mk_v[r, sl] > THRESH
                    accs[6 + p] = accs[6 + p] + jnp.where(m, 1.0, 0.0)
                    for c, (av, bv) in enumerate(((a0, b0), (a1, b1), (a2, b2))):
                        d = jnp.abs(av[r, sl] - bv[r, sl])
                        accs[2 * c + p] = accs[2 * c + p] + jnp.where(m, d, 0.0)
                return tuple(accs)

            return lax.fori_loop(0, W2D // (L * U), step, carry)

        carry0 = lax.fori_loop(0, RT, row_loop, carry0)

    sum_b[...] = (carry0[0] + carry0[1]) + (carry0[2] + carry0[3]) \
        + (carry0[4] + carry0[5])
    cnt_b[...] = carry0[6] + carry0[7]
    pltpu.sync_copy(sum_b, out_hbm.at[0, wid])
    pltpu.sync_copy(cnt_b, out_hbm.at[1, wid])


BB = 1


def _tc_body(in_ref, tg_ref, mk_ref, out_ref):
    g = pl.program_id(0)
    psum = jnp.zeros((8, W), jnp.float32)
    pcnt = jnp.zeros((8, W), jnp.float32)
    for k in range(BB):
        mf = jnp.where(mk_ref[k, 0] > THRESH, 1.0, 0.0)
        a, t = in_ref[k], tg_ref[k]
        dsum = (jnp.abs(a[0] - t[0]) + jnp.abs(a[1] - t[1])) \
            + jnp.abs(a[2] - t[2])
        masked = dsum * mf
        psum = psum + jnp.sum(masked.reshape(H // 8, 8, W), axis=0)
        pcnt = pcnt + jnp.sum(mf.reshape(H // 8, 8, W), axis=0)

    @pl.when(g == 0)
    def _init():
        out_ref[0] = psum
        out_ref[1] = pcnt

    @pl.when(g > 0)
    def _accum():
        out_ref[0] += psum
        out_ref[1] += pcnt


def _tc_partials(input, target, mask):
    return pl.pallas_call(
        _tc_body,
        grid=((B - K_SC) // BB,),
        in_specs=[
            pl.BlockSpec((BB, C, H, W), lambda g: (g + K_SC // BB, 0, 0, 0)),
            pl.BlockSpec((BB, C, H, W), lambda g: (g + K_SC // BB, 0, 0, 0)),
            pl.BlockSpec((BB, 1, H, W), lambda g: (g + K_SC // BB, 0, 0, 0)),
        ],
        out_specs=pl.BlockSpec((2, 8, W), lambda g: (0, 0, 0)),
        out_shape=jax.ShapeDtypeStruct((2, 8, W), jnp.float32),
    )(input, target, mask)


def kernel(input, target, mask):
    sc = _sc_partials(
        input.reshape(B * C * H, W),
        target.reshape(B * C * H, W),
        mask.reshape(B * H, W))
    tc = _tc_partials(input, target, mask)
    sel_sum = jnp.sum(sc[0]) + jnp.sum(tc[0])
    count = C * (jnp.sum(sc[1]) + jnp.sum(tc[1]))
    return sel_sum / jnp.maximum(count, 1.0)
